# Initial kernel scaffold; baseline (speedup 1.0000x reference)
import jax
import jax.numpy as jnp
from jax.experimental import pallas as pl


def kernel(x, edge_index, W):
    raise NotImplementedError("write your pallas kernel here")


# speedup vs baseline: 6.3062x; 6.3062x over previous
import functools

import jax
import jax.numpy as jnp
from jax import lax
from jax.experimental import pallas as pl
from jax.experimental.pallas import tpu as pltpu
from jax.experimental.pallas import tpu_sc as plsc

CHUNK = 128
NSUB = 16
NCORE = 2
LANES = 16


def _sc_edge_pass(n_acc, cpt, dh):
    def body(xl_hbm, xr_hbm, src_hbm, dst_hbm,
             suml_hbm, sumr_hbm, deg_hbm,
             acc_sh, deg_sh, src_t, dst_t, rows, ones, z1d):
        cid = lax.axis_index("c")
        sid = lax.axis_index("s")

        def _zrow(i, _):
            def _z16(k, _):
                rows[i, pl.ds(k * LANES, LANES)] = jnp.zeros((LANES,), jnp.float32)
                return 0
            return lax.fori_loop(0, dh // LANES, _z16, 0)
        lax.fori_loop(0, CHUNK, _zrow, 0)

        def _z1(k, _):
            z1d[pl.ds(k * LANES, LANES)] = jnp.zeros((LANES,), jnp.float32)
            return 0
        lax.fori_loop(0, (n_acc // NSUB) // LANES, _z1, 0)

        def _o1(k, _):
            ones[pl.ds(k * LANES, LANES)] = jnp.ones((LANES,), jnp.float32)
            return 0
        lax.fori_loop(0, CHUNK // LANES, _o1, 0)

        rows_per_tile = n_acc // NSUB
        row0 = sid * rows_per_tile
        nz = rows_per_tile // CHUNK
        for i in range(nz):
            pltpu.sync_copy(rows, acc_sh.at[pl.ds(row0 + i * CHUNK, CHUNK)])
        pltpu.sync_copy(z1d, deg_sh.at[pl.ds(row0, rows_per_tile)])
        plsc.subcore_barrier()

        pltpu.sync_copy(src_hbm.at[sid], src_t)
        pltpu.sync_copy(dst_hbm.at[sid], dst_t)

        def edge_loop(xh_hbm):
            def _chunk(j, _):
                pltpu.sync_copy(xh_hbm.at[src_t.at[j]], rows)
                pltpu.sync_copy(rows, acc_sh.at[dst_t.at[j]], add=True)
                pltpu.sync_copy(ones, deg_sh.at[dst_t.at[j]], add=True)
                return 0
            lax.fori_loop(0, cpt, _chunk, 0)

        pl.when(cid == 0)(lambda: edge_loop(xl_hbm))
        pl.when(cid == 1)(lambda: edge_loop(xr_hbm))
        plsc.subcore_barrier()

        def wout(out_hbm):
            pltpu.sync_copy(acc_sh.at[pl.ds(row0, rows_per_tile)],
                            out_hbm.at[pl.ds(row0, rows_per_tile)])

        pl.when(cid == 0)(lambda: wout(suml_hbm))
        pl.when(cid == 1)(lambda: wout(sumr_hbm))
        pl.when(cid == 0)(lambda: pltpu.sync_copy(
            deg_sh.at[pl.ds(row0, rows_per_tile)],
            deg_hbm.at[pl.ds(row0, rows_per_tile)]))

    return body


def _tc_node_pass(xl_ref, xr_ref, sl_ref, sr_ref, deg_ref, w_ref,
                  q_ref, base_ref, invd_ref):
    invd = 1.0 / jnp.maximum(deg_ref[...], 1.0)
    w1l = w_ref[:, 0:128]
    w1r = w_ref[:, 128:256]
    w2l = w_ref[:, 256:384]
    w2r = w_ref[:, 384:512]
    w3l = w_ref[:, 512:640]
    w3r = w_ref[:, 640:768]
    xl = xl_ref[...]
    xr = xr_ref[...]
    ml = sl_ref[...] * invd
    mr = sr_ref[...] * invd
    dl = xl - ml
    dr = xr - mr
    q = (jnp.sum(w2l * dl * dl, axis=1, keepdims=True) +
         jnp.sum(w2r * dr * dr, axis=1, keepdims=True))
    base = (jnp.sum(w1l * ml + w3l * xl, axis=1, keepdims=True) +
            jnp.sum(w1r * mr + w3r * xr, axis=1, keepdims=True))
    q_ref[...] = q
    base_ref[...] = base
    invd_ref[...] = invd


def _sc_scalar_pass(n_acc, cpt):
    def body(q_hbm, src_hbm, dst_hbm, base_hbm, invd_hbm,
             score_hbm, s2_sh, src_t, dst_t, vals, z1d, sv, bv, iv):
        cid = lax.axis_index("c")
        sid = lax.axis_index("s")
        rows_per_tile = n_acc // NSUB
        row0 = sid * rows_per_tile

        @pl.when(cid == 0)
        def _():
            def _z1(k, _):
                z1d[pl.ds(k * LANES, LANES)] = jnp.zeros((LANES,), jnp.float32)
                return 0
            lax.fori_loop(0, rows_per_tile // LANES, _z1, 0)
            pltpu.sync_copy(z1d, s2_sh.at[pl.ds(row0, rows_per_tile)])
            plsc.subcore_barrier()

            pltpu.sync_copy(src_hbm.at[sid], src_t)
            pltpu.sync_copy(dst_hbm.at[sid], dst_t)

            def _chunk(j, _):
                pltpu.sync_copy(q_hbm.at[src_t.at[j]], vals)
                pltpu.sync_copy(vals, s2_sh.at[dst_t.at[j]], add=True)
                return 0
            lax.fori_loop(0, cpt, _chunk, 0)
            plsc.subcore_barrier()

            pltpu.sync_copy(s2_sh.at[pl.ds(row0, rows_per_tile)], sv)
            pltpu.sync_copy(base_hbm.at[pl.ds(row0, rows_per_tile)], bv)
            pltpu.sync_copy(invd_hbm.at[pl.ds(row0, rows_per_tile)], iv)

            def _node(k, _):
                sl = pl.ds(k * LANES, LANES)
                z = bv[sl] + sv[sl] * iv[sl]
                sv[sl] = 1.0 / (1.0 + jnp.exp(-z))
                return 0
            lax.fori_loop(0, rows_per_tile // LANES, _node, 0)
            pltpu.sync_copy(sv, score_hbm.at[pl.ds(row0, rows_per_tile)])

    return body


@jax.jit
def kernel(x, edge_index, W):
    n, d = x.shape
    e = edge_index.shape[1]
    dh = d // 2

    cpt = -(-e // (NSUB * CHUNK))
    e_pad = NSUB * CHUNK * cpt
    n_acc = -(-(n + CHUNK) // (NSUB * CHUNK)) * (NSUB * CHUNK)

    pad = e_pad - e
    src = jnp.concatenate([edge_index[0], jnp.zeros((pad,), jnp.int32)])
    dst = jnp.concatenate(
        [edge_index[1],
         n + (jnp.arange(pad, dtype=jnp.int32) % CHUNK)])
    src_r = src.reshape(NSUB, cpt, CHUNK)
    dst_r = dst.reshape(NSUB, cpt, CHUNK)
    xl = x[:, :dh]
    xr = x[:, dh:]

    mesh = plsc.VectorSubcoreMesh(core_axis_name="c", subcore_axis_name="s")

    edge_kernel = pl.kernel(
        _sc_edge_pass(n_acc, cpt, dh),
        out_type=[
            jax.ShapeDtypeStruct((n_acc, dh), jnp.float32),
            jax.ShapeDtypeStruct((n_acc, dh), jnp.float32),
            jax.ShapeDtypeStruct((n_acc,), jnp.float32),
        ],
        mesh=mesh,
        scratch_types=[
            pltpu.MemorySpace.VMEM_SHARED((n_acc, dh), jnp.float32),
            pltpu.MemorySpace.VMEM_SHARED((n_acc,), jnp.float32),
            pltpu.VMEM((cpt, CHUNK), jnp.int32),
            pltpu.VMEM((cpt, CHUNK), jnp.int32),
            pltpu.VMEM((CHUNK, dh), jnp.float32),
            pltpu.VMEM((CHUNK,), jnp.float32),
            pltpu.VMEM((n_acc // NSUB,), jnp.float32),
        ],
    )
    suml, sumr, deg = edge_kernel(xl, xr, src_r, dst_r)

    nb = 400
    grid = n // nb
    q, base, invd = pl.pallas_call(
        _tc_node_pass,
        grid=(grid,),
        in_specs=[
            pl.BlockSpec((nb, dh), lambda i: (i, 0)),
            pl.BlockSpec((nb, dh), lambda i: (i, 0)),
            pl.BlockSpec((nb, dh), lambda i: (i, 0)),
            pl.BlockSpec((nb, dh), lambda i: (i, 0)),
            pl.BlockSpec((nb, 1), lambda i: (i, 0)),
            pl.BlockSpec((1, 3 * d), lambda i: (0, 0)),
        ],
        out_specs=[
            pl.BlockSpec((nb, 1), lambda i: (i, 0)),
            pl.BlockSpec((nb, 1), lambda i: (i, 0)),
            pl.BlockSpec((nb, 1), lambda i: (i, 0)),
        ],
        out_shape=[
            jax.ShapeDtypeStruct((n, 1), jnp.float32),
            jax.ShapeDtypeStruct((n, 1), jnp.float32),
            jax.ShapeDtypeStruct((n, 1), jnp.float32),
        ],
    )(xl, xr, suml[:n], sumr[:n], deg[:n, None], W)

    zpad = jnp.zeros((n_acc - n,), jnp.float32)
    base_p = jnp.concatenate([base[:, 0], zpad])
    invd_p = jnp.concatenate([invd[:, 0], zpad])
    rows_per_tile = n_acc // NSUB
    scalar_kernel = pl.kernel(
        _sc_scalar_pass(n_acc, cpt),
        out_type=jax.ShapeDtypeStruct((n_acc,), jnp.float32),
        mesh=mesh,
        scratch_types=[
            pltpu.MemorySpace.VMEM_SHARED((n_acc,), jnp.float32),
            pltpu.VMEM((cpt, CHUNK), jnp.int32),
            pltpu.VMEM((cpt, CHUNK), jnp.int32),
            pltpu.VMEM((CHUNK,), jnp.float32),
            pltpu.VMEM((rows_per_tile,), jnp.float32),
            pltpu.VMEM((rows_per_tile,), jnp.float32),
            pltpu.VMEM((rows_per_tile,), jnp.float32),
            pltpu.VMEM((rows_per_tile,), jnp.float32),
        ],
    )
    score = scalar_kernel(q[:, 0], src_r, dst_r, base_p, invd_p)
    return score[:n, None]


# speedup vs baseline: 6.3714x; 1.0103x over previous
import jax
import jax.numpy as jnp
from jax import lax
from jax.experimental import pallas as pl
from jax.experimental.pallas import tpu as pltpu
from jax.experimental.pallas import tpu_sc as plsc

CHUNK = 128
NSUB = 16
NPHASE = 2
LANES = 16


def _sc_edge_pass(n_acc, half, dh):
    def body(xl_hbm, xr_hbm, src_hbm, dst_hbm,
             suml_hbm, sumr_hbm, deg_hbm,
             acc_sh, deg_sh, src_t, dst_t, r0, r1, ones, z1d,
             gsem, asem, dsem):
        cid = lax.axis_index("c")
        sid = lax.axis_index("s")
        rows = [r0, r1]

        def _zrow(i, _):
            def _z16(k, _):
                r0[i, pl.ds(k * LANES, LANES)] = jnp.zeros((LANES,), jnp.float32)
                return 0
            return lax.fori_loop(0, dh // LANES, _z16, 0)
        lax.fori_loop(0, CHUNK, _zrow, 0)

        def _z1(k, _):
            z1d[pl.ds(k * LANES, LANES)] = jnp.zeros((LANES,), jnp.float32)
            return 0
        lax.fori_loop(0, (n_acc // NSUB) // LANES, _z1, 0)

        def _o1(k, _):
            ones[pl.ds(k * LANES, LANES)] = jnp.ones((LANES,), jnp.float32)
            return 0
        lax.fori_loop(0, CHUNK // LANES, _o1, 0)

        zrows = n_acc // NSUB
        zrow0 = sid * zrows
        for i in range(zrows // CHUNK):
            pltpu.sync_copy(r0, acc_sh.at[pl.ds(zrow0 + i * CHUNK, CHUNK)])
        zrem = zrows % CHUNK
        if zrem:
            pltpu.sync_copy(
                r0.at[pl.ds(0, zrem)],
                acc_sh.at[pl.ds(zrow0 + zrows - zrem, zrem)])
        pltpu.sync_copy(z1d, deg_sh.at[pl.ds(zrow0, zrows)])
        plsc.subcore_barrier()

        def edge_loop(xh_hbm, do_deg):
            for ph in range(NPHASE):
                tid = sid * NPHASE + ph
                pltpu.sync_copy(src_hbm.at[tid], src_t)
                pltpu.sync_copy(dst_hbm.at[tid], dst_t)
                pltpu.async_copy(xh_hbm.at[src_t.at[0]], r0, gsem.at[0])

                def chunk(j, b):
                    rb = rows[b]
                    ro = rows[1 - b]
                    @pl.when(j >= 1)
                    def _():
                        pltpu.make_async_copy(ro,
                                              acc_sh.at[dst_t.at[j - 1]],
                                              asem.at[1 - b]).wait()
                        if do_deg:
                            pltpu.make_async_copy(
                                ones, deg_sh.at[dst_t.at[j - 1]],
                                dsem.at[1 - b]).wait()

                    @pl.when(j + 1 < half)
                    def _():
                        pltpu.async_copy(xh_hbm.at[src_t.at[j + 1]], ro,
                                         gsem.at[1 - b])

                    pltpu.make_async_copy(xh_hbm.at[src_t.at[j]], rb,
                                          gsem.at[b]).wait()
                    pltpu.async_copy(rb, acc_sh.at[dst_t.at[j]], asem.at[b],
                                     add=True)
                    if do_deg:
                        pltpu.async_copy(ones, deg_sh.at[dst_t.at[j]],
                                         dsem.at[b], add=True)

                def it_body(it, _):
                    chunk(it * 2, 0)
                    chunk(it * 2 + 1, 1)
                    return 0
                lax.fori_loop(0, half // 2, it_body, 0)

                pltpu.make_async_copy(r1, acc_sh.at[dst_t.at[half - 1]],
                                      asem.at[1]).wait()
                if do_deg:
                    pltpu.make_async_copy(ones,
                                          deg_sh.at[dst_t.at[half - 1]],
                                          dsem.at[1]).wait()

        pl.when(cid == 0)(lambda: edge_loop(xl_hbm, True))
        pl.when(cid == 1)(lambda: edge_loop(xr_hbm, False))
        plsc.subcore_barrier()

        def wout(out_hbm):
            pltpu.sync_copy(acc_sh.at[pl.ds(zrow0, zrows)],
                            out_hbm.at[pl.ds(zrow0, zrows)])

        pl.when(cid == 0)(lambda: wout(suml_hbm))
        pl.when(cid == 1)(lambda: wout(sumr_hbm))
        pl.when(cid == 0)(lambda: pltpu.sync_copy(
            deg_sh.at[pl.ds(zrow0, zrows)],
            deg_hbm.at[pl.ds(zrow0, zrows)]))

    return body


def _tc_node_pass(xl_ref, xr_ref, sl_ref, sr_ref, deg_ref, w_ref,
                  q_ref, base_ref, invd_ref):
    invd = 1.0 / jnp.maximum(deg_ref[...], 1.0)
    w1l = w_ref[:, 0:128]
    w1r = w_ref[:, 128:256]
    w2l = w_ref[:, 256:384]
    w2r = w_ref[:, 384:512]
    w3l = w_ref[:, 512:640]
    w3r = w_ref[:, 640:768]
    xl = xl_ref[...]
    xr = xr_ref[...]
    ml = sl_ref[...] * invd
    mr = sr_ref[...] * invd
    dl = xl - ml
    dr = xr - mr
    q = (jnp.sum(w2l * dl * dl, axis=1, keepdims=True) +
         jnp.sum(w2r * dr * dr, axis=1, keepdims=True))
    base = (jnp.sum(w1l * ml + w3l * xl, axis=1, keepdims=True) +
            jnp.sum(w1r * mr + w3r * xr, axis=1, keepdims=True))
    q_ref[...] = q
    base_ref[...] = base
    invd_ref[...] = invd


def _sc_scalar_pass(n, n_acc, half):
    def body(q_hbm, src_hbm, dst_hbm, base_hbm, invd_hbm,
             score_hbm, s2_sh, src_t, dst_t, v0, v1, z1d,
             sv, bv, iv, gsem, ssem):
        cid = lax.axis_index("c")
        sid = lax.axis_index("s")
        rows_per_tile = n_acc // NSUB
        row0 = sid * rows_per_tile
        vals = [v0, v1]

        @pl.when(cid == 0)
        def _():
            def _z1(k, _):
                z1d[pl.ds(k * LANES, LANES)] = jnp.zeros((LANES,), jnp.float32)
                return 0
            lax.fori_loop(0, rows_per_tile // LANES, _z1, 0)
            pltpu.sync_copy(z1d, s2_sh.at[pl.ds(row0, rows_per_tile)])
            plsc.subcore_barrier()

            for ph in range(NPHASE):
                tid = sid * NPHASE + ph
                pltpu.sync_copy(src_hbm.at[tid], src_t)
                pltpu.sync_copy(dst_hbm.at[tid], dst_t)
                pltpu.async_copy(q_hbm.at[src_t.at[0]], v0, gsem.at[0])

                def chunk(j, b):
                    vb = vals[b]
                    vo = vals[1 - b]
                    @pl.when(j >= 1)
                    def _():
                        pltpu.make_async_copy(vo, s2_sh.at[dst_t.at[j - 1]],
                                              ssem.at[1 - b]).wait()

                    @pl.when(j + 1 < half)
                    def _():
                        pltpu.async_copy(q_hbm.at[src_t.at[j + 1]], vo,
                                         gsem.at[1 - b])

                    pltpu.make_async_copy(q_hbm.at[src_t.at[j]], vb,
                                          gsem.at[b]).wait()
                    pltpu.async_copy(vb, s2_sh.at[dst_t.at[j]], ssem.at[b],
                                     add=True)

                def it_body(it, _):
                    chunk(it * 2, 0)
                    chunk(it * 2 + 1, 1)
                    return 0
                lax.fori_loop(0, half // 2, it_body, 0)
                pltpu.make_async_copy(v1, s2_sh.at[dst_t.at[half - 1]],
                                      ssem.at[1]).wait()
            plsc.subcore_barrier()

            pltpu.sync_copy(s2_sh.at[pl.ds(row0, rows_per_tile)], sv)
            pltpu.sync_copy(base_hbm.at[pl.ds(row0, rows_per_tile)], bv)
            pltpu.sync_copy(invd_hbm.at[pl.ds(row0, rows_per_tile)], iv)

            def _node(k, _):
                sl = pl.ds(k * LANES, LANES)
                z = bv[sl] + sv[sl] * iv[sl]
                sv[sl] = 1.0 / (1.0 + jnp.exp(-z))
                return 0
            lax.fori_loop(0, rows_per_tile // LANES, _node, 0)
            pltpu.sync_copy(sv, score_hbm.at[pl.ds(row0, rows_per_tile)])

    return body


@jax.jit
def kernel(x, edge_index, W):
    n, d = x.shape
    e = edge_index.shape[1]
    dh = d // 2

    half = -(-e // (NSUB * CHUNK * NPHASE * 2)) * 2
    cpt = NPHASE * half
    e_pad = NSUB * CHUNK * cpt
    n_acc = -(-(n + CHUNK) // (NSUB * 128)) * (NSUB * 128)

    pad = e_pad - e
    src = jnp.concatenate([edge_index[0], jnp.zeros((pad,), jnp.int32)])
    dst = jnp.concatenate(
        [edge_index[1],
         n + (jnp.arange(pad, dtype=jnp.int32) % CHUNK)])
    src_r = src.reshape(NSUB * NPHASE, half, CHUNK)
    dst_r = dst.reshape(NSUB * NPHASE, half, CHUNK)
    xl = x[:, :dh]
    xr = x[:, dh:]

    mesh = plsc.VectorSubcoreMesh(core_axis_name="c", subcore_axis_name="s")

    edge_kernel = pl.kernel(
        _sc_edge_pass(n_acc, half, dh),
        out_type=[
            jax.ShapeDtypeStruct((n_acc, dh), jnp.float32),
            jax.ShapeDtypeStruct((n_acc, dh), jnp.float32),
            jax.ShapeDtypeStruct((n_acc,), jnp.float32),
        ],
        mesh=mesh,
        scratch_types=[
            pltpu.MemorySpace.VMEM_SHARED((n_acc, dh), jnp.float32),
            pltpu.MemorySpace.VMEM_SHARED((n_acc,), jnp.float32),
            pltpu.VMEM((half, CHUNK), jnp.int32),
            pltpu.VMEM((half, CHUNK), jnp.int32),
            pltpu.VMEM((CHUNK, dh), jnp.float32),
            pltpu.VMEM((CHUNK, dh), jnp.float32),
            pltpu.VMEM((CHUNK,), jnp.float32),
            pltpu.VMEM((n_acc // NSUB,), jnp.float32),
            pltpu.SemaphoreType.DMA((2,)),
            pltpu.SemaphoreType.DMA((2,)),
            pltpu.SemaphoreType.DMA((2,)),
        ],
    )
    suml, sumr, deg = edge_kernel(xl, xr, src_r, dst_r)

    nb = 400
    grid = n // nb
    q, base, invd = pl.pallas_call(
        _tc_node_pass,
        grid=(grid,),
        in_specs=[
            pl.BlockSpec((nb, dh), lambda i: (i, 0)),
            pl.BlockSpec((nb, dh), lambda i: (i, 0)),
            pl.BlockSpec((nb, dh), lambda i: (i, 0)),
            pl.BlockSpec((nb, dh), lambda i: (i, 0)),
            pl.BlockSpec((nb, 1), lambda i: (i, 0)),
            pl.BlockSpec((1, 3 * d), lambda i: (0, 0)),
        ],
        out_specs=[
            pl.BlockSpec((nb, 1), lambda i: (i, 0)),
            pl.BlockSpec((nb, 1), lambda i: (i, 0)),
            pl.BlockSpec((nb, 1), lambda i: (i, 0)),
        ],
        out_shape=[
            jax.ShapeDtypeStruct((n, 1), jnp.float32),
            jax.ShapeDtypeStruct((n, 1), jnp.float32),
            jax.ShapeDtypeStruct((n, 1), jnp.float32),
        ],
    )(xl, xr, suml, sumr, deg.reshape(n_acc, 1), W)

    zpad = jnp.zeros((n_acc - n,), jnp.float32)
    q_p = jnp.concatenate([q[:, 0], zpad])
    base_p = jnp.concatenate([base[:, 0], zpad])
    invd_p = jnp.concatenate([invd[:, 0], zpad])
    rows_per_tile = n_acc // NSUB
    scalar_kernel = pl.kernel(
        _sc_scalar_pass(n, n_acc, half),
        out_type=jax.ShapeDtypeStruct((n_acc,), jnp.float32),
        mesh=mesh,
        scratch_types=[
            pltpu.MemorySpace.VMEM_SHARED((n_acc,), jnp.float32),
            pltpu.VMEM((half, CHUNK), jnp.int32),
            pltpu.VMEM((half, CHUNK), jnp.int32),
            pltpu.VMEM((CHUNK,), jnp.float32),
            pltpu.VMEM((CHUNK,), jnp.float32),
            pltpu.VMEM((rows_per_tile,), jnp.float32),
            pltpu.VMEM((rows_per_tile,), jnp.float32),
            pltpu.VMEM((rows_per_tile,), jnp.float32),
            pltpu.VMEM((rows_per_tile,), jnp.float32),
            pltpu.SemaphoreType.DMA((2,)),
            pltpu.SemaphoreType.DMA((2,)),
        ],
    )
    score = scalar_kernel(q_p, src_r, dst_r, base_p, invd_p)
    return score[:n, None]


# speedup vs baseline: 9.2797x; 1.4565x over previous
import jax
import jax.numpy as jnp
from jax import lax
from jax.experimental import pallas as pl
from jax.experimental.pallas import tpu as pltpu
from jax.experimental.pallas import tpu_sc as plsc

CHUNK = 80
NSUB = 16
NPHASE = 2
LANES = 16


def _sc_edge_pass(n_acc, half, dh):
    def body(xl_hbm, xr_hbm, src_hbm, dst_hbm,
             suml_hbm, sumr_hbm, deg_hbm,
             acc_sh, deg_sh, src_t, dst_t, r0, r1, r2, ones, z1d,
             gsem, asem, dsem):
        cid = lax.axis_index("c")
        sid = lax.axis_index("s")
        rows = [r0, r1, r2]

        def _zrow(i, _):
            def _z16(k, _):
                r0[i, pl.ds(k * LANES, LANES)] = jnp.zeros((LANES,), jnp.float32)
                return 0
            return lax.fori_loop(0, dh // LANES, _z16, 0)
        lax.fori_loop(0, CHUNK, _zrow, 0)

        def _z1(k, _):
            z1d[pl.ds(k * LANES, LANES)] = jnp.zeros((LANES,), jnp.float32)
            return 0
        lax.fori_loop(0, (n_acc // NSUB) // LANES, _z1, 0)

        def _o1(k, _):
            ones[pl.ds(k * LANES, LANES)] = jnp.ones((LANES,), jnp.float32)
            return 0
        lax.fori_loop(0, CHUNK // LANES, _o1, 0)

        zrows = n_acc // NSUB
        zrow0 = sid * zrows
        for i in range(zrows // CHUNK):
            pltpu.sync_copy(r0, acc_sh.at[pl.ds(zrow0 + i * CHUNK, CHUNK)])
        zrem = zrows % CHUNK
        if zrem:
            pltpu.sync_copy(
                r0.at[pl.ds(0, zrem)],
                acc_sh.at[pl.ds(zrow0 + zrows - zrem, zrem)])
        pltpu.sync_copy(z1d, deg_sh.at[pl.ds(zrow0, zrows)])
        plsc.subcore_barrier()

        def edge_loop(xh_hbm, do_deg):
            for ph in range(NPHASE):
                tid = sid * NPHASE + ph
                pltpu.sync_copy(src_hbm.at[tid], src_t)
                pltpu.sync_copy(dst_hbm.at[tid], dst_t)
                pltpu.async_copy(xh_hbm.at[src_t.at[0]], r0, gsem.at[0])
                pltpu.async_copy(xh_hbm.at[src_t.at[1]], r1, gsem.at[1])

                def chunk(j, b):
                    rb = rows[b]
                    bn = (b + 2) % 3
                    pltpu.make_async_copy(xh_hbm.at[src_t.at[j]], rb,
                                          gsem.at[b]).wait()
                    pltpu.async_copy(rb, acc_sh.at[dst_t.at[j]], asem.at[b],
                                     add=True)
                    if do_deg:
                        pltpu.async_copy(ones, deg_sh.at[dst_t.at[j]],
                                         dsem.at[b], add=True)

                    @pl.when(j >= 1)
                    def _():
                        pltpu.make_async_copy(rows[bn],
                                              acc_sh.at[dst_t.at[j - 1]],
                                              asem.at[bn]).wait()
                        if do_deg:
                            pltpu.make_async_copy(
                                ones, deg_sh.at[dst_t.at[j - 1]],
                                dsem.at[bn]).wait()

                    @pl.when(j + 2 < half)
                    def _():
                        pltpu.async_copy(xh_hbm.at[src_t.at[j + 2]],
                                         rows[bn], gsem.at[bn])

                def it_body(it, _):
                    for k in range(3):
                        chunk(it * 3 + k, k)
                    return 0
                lax.fori_loop(0, half // 3, it_body, 0)

                pltpu.make_async_copy(r2, acc_sh.at[dst_t.at[half - 1]],
                                      asem.at[2]).wait()
                if do_deg:
                    pltpu.make_async_copy(ones,
                                          deg_sh.at[dst_t.at[half - 1]],
                                          dsem.at[2]).wait()

        pl.when(cid == 0)(lambda: edge_loop(xl_hbm, True))
        pl.when(cid == 1)(lambda: edge_loop(xr_hbm, False))
        plsc.subcore_barrier()

        def wout(out_hbm):
            pltpu.sync_copy(acc_sh.at[pl.ds(zrow0, zrows)],
                            out_hbm.at[pl.ds(zrow0, zrows)])

        pl.when(cid == 0)(lambda: wout(suml_hbm))
        pl.when(cid == 1)(lambda: wout(sumr_hbm))
        pl.when(cid == 0)(lambda: pltpu.sync_copy(
            deg_sh.at[pl.ds(zrow0, zrows)],
            deg_hbm.at[pl.ds(zrow0, zrows)]))

    return body


def _tc_node_pass(xl_ref, xr_ref, sl_ref, sr_ref, deg_ref, w_ref,
                  q_ref, base_ref, invd_ref):
    invd = 1.0 / jnp.maximum(deg_ref[...], 1.0)
    w1l = w_ref[:, 0:128]
    w1r = w_ref[:, 128:256]
    w2l = w_ref[:, 256:384]
    w2r = w_ref[:, 384:512]
    w3l = w_ref[:, 512:640]
    w3r = w_ref[:, 640:768]
    xl = xl_ref[...]
    xr = xr_ref[...]
    ml = sl_ref[...] * invd
    mr = sr_ref[...] * invd
    dl = xl - ml
    dr = xr - mr
    q = (jnp.sum(w2l * dl * dl, axis=1, keepdims=True) +
         jnp.sum(w2r * dr * dr, axis=1, keepdims=True))
    base = (jnp.sum(w1l * ml + w3l * xl, axis=1, keepdims=True) +
            jnp.sum(w1r * mr + w3r * xr, axis=1, keepdims=True))
    q_ref[...] = q
    base_ref[...] = base
    invd_ref[...] = invd


def _sc_scalar_pass(n, n_acc, half):
    def body(q_hbm, src_hbm, dst_hbm, base_hbm, invd_hbm,
             score_hbm, s2_sh, src_t, dst_t, v0, v1, v2, z1d,
             sv, bv, iv, gsem, ssem):
        cid = lax.axis_index("c")
        sid = lax.axis_index("s")
        rows_per_tile = n_acc // NSUB
        row0 = sid * rows_per_tile
        vals = [v0, v1, v2]

        @pl.when(cid == 0)
        def _():
            def _z1(k, _):
                z1d[pl.ds(k * LANES, LANES)] = jnp.zeros((LANES,), jnp.float32)
                return 0
            lax.fori_loop(0, rows_per_tile // LANES, _z1, 0)
            pltpu.sync_copy(z1d, s2_sh.at[pl.ds(row0, rows_per_tile)])
            plsc.subcore_barrier()

            for ph in range(NPHASE):
                tid = sid * NPHASE + ph
                pltpu.sync_copy(src_hbm.at[tid], src_t)
                pltpu.sync_copy(dst_hbm.at[tid], dst_t)
                pltpu.async_copy(q_hbm.at[src_t.at[0]], v0, gsem.at[0])
                pltpu.async_copy(q_hbm.at[src_t.at[1]], v1, gsem.at[1])

                def chunk(j, b):
                    vb = vals[b]
                    bn = (b + 2) % 3
                    pltpu.make_async_copy(q_hbm.at[src_t.at[j]], vb,
                                          gsem.at[b]).wait()
                    pltpu.async_copy(vb, s2_sh.at[dst_t.at[j]], ssem.at[b],
                                     add=True)

                    @pl.when(j >= 1)
                    def _():
                        pltpu.make_async_copy(vals[bn],
                                              s2_sh.at[dst_t.at[j - 1]],
                                              ssem.at[bn]).wait()

                    @pl.when(j + 2 < half)
                    def _():
                        pltpu.async_copy(q_hbm.at[src_t.at[j + 2]],
                                         vals[bn], gsem.at[bn])

                def it_body(it, _):
                    for k in range(3):
                        chunk(it * 3 + k, k)
                    return 0
                lax.fori_loop(0, half // 3, it_body, 0)
                pltpu.make_async_copy(v2, s2_sh.at[dst_t.at[half - 1]],
                                      ssem.at[2]).wait()
            plsc.subcore_barrier()

            pltpu.sync_copy(s2_sh.at[pl.ds(row0, rows_per_tile)], sv)
            pltpu.sync_copy(base_hbm.at[pl.ds(row0, rows_per_tile)], bv)
            pltpu.sync_copy(invd_hbm.at[pl.ds(row0, rows_per_tile)], iv)

            def _node(k, _):
                sl = pl.ds(k * LANES, LANES)
                z = bv[sl] + sv[sl] * iv[sl]
                sv[sl] = 1.0 / (1.0 + jnp.exp(-z))
                return 0
            lax.fori_loop(0, rows_per_tile // LANES, _node, 0)
            pltpu.sync_copy(sv, score_hbm.at[pl.ds(row0, rows_per_tile)])

    return body


@jax.jit
def kernel(x, edge_index, W):
    n, d = x.shape
    e = edge_index.shape[1]
    dh = d // 2

    half = -(-e // (NSUB * CHUNK * NPHASE * 3)) * 3
    cpt = NPHASE * half
    e_pad = NSUB * CHUNK * cpt
    n_acc = -(-(n + CHUNK) // (NSUB * 128)) * (NSUB * 128)

    pad = e_pad - e
    src = jnp.concatenate([edge_index[0], jnp.zeros((pad,), jnp.int32)])
    dst = jnp.concatenate(
        [edge_index[1],
         n + (jnp.arange(pad, dtype=jnp.int32) % CHUNK)])
    src_r = src.reshape(NSUB * NPHASE, half, CHUNK)
    dst_r = dst.reshape(NSUB * NPHASE, half, CHUNK)
    xl = x[:, :dh]
    xr = x[:, dh:]

    mesh = plsc.VectorSubcoreMesh(core_axis_name="c", subcore_axis_name="s")

    edge_kernel = pl.kernel(
        _sc_edge_pass(n_acc, half, dh),
        out_type=[
            jax.ShapeDtypeStruct((n_acc, dh), jnp.float32),
            jax.ShapeDtypeStruct((n_acc, dh), jnp.float32),
            jax.ShapeDtypeStruct((n_acc,), jnp.float32),
        ],
        mesh=mesh,
        scratch_types=[
            pltpu.MemorySpace.VMEM_SHARED((n_acc, dh), jnp.float32),
            pltpu.MemorySpace.VMEM_SHARED((n_acc,), jnp.float32),
            pltpu.VMEM((half, CHUNK), jnp.int32),
            pltpu.VMEM((half, CHUNK), jnp.int32),
            pltpu.VMEM((CHUNK, dh), jnp.float32),
            pltpu.VMEM((CHUNK, dh), jnp.float32),
            pltpu.VMEM((CHUNK, dh), jnp.float32),
            pltpu.VMEM((CHUNK,), jnp.float32),
            pltpu.VMEM((n_acc // NSUB,), jnp.float32),
            pltpu.SemaphoreType.DMA((3,)),
            pltpu.SemaphoreType.DMA((3,)),
            pltpu.SemaphoreType.DMA((3,)),
        ],
    )
    suml, sumr, deg = edge_kernel(xl, xr, src_r, dst_r)

    nb = 400
    grid = n // nb
    q, base, invd = pl.pallas_call(
        _tc_node_pass,
        grid=(grid,),
        in_specs=[
            pl.BlockSpec((nb, dh), lambda i: (i, 0)),
            pl.BlockSpec((nb, dh), lambda i: (i, 0)),
            pl.BlockSpec((nb, dh), lambda i: (i, 0)),
            pl.BlockSpec((nb, dh), lambda i: (i, 0)),
            pl.BlockSpec((nb, 1), lambda i: (i, 0)),
            pl.BlockSpec((1, 3 * d), lambda i: (0, 0)),
        ],
        out_specs=[
            pl.BlockSpec((nb, 1), lambda i: (i, 0)),
            pl.BlockSpec((nb, 1), lambda i: (i, 0)),
            pl.BlockSpec((nb, 1), lambda i: (i, 0)),
        ],
        out_shape=[
            jax.ShapeDtypeStruct((n, 1), jnp.float32),
            jax.ShapeDtypeStruct((n, 1), jnp.float32),
            jax.ShapeDtypeStruct((n, 1), jnp.float32),
        ],
    )(xl, xr, suml, sumr, deg.reshape(n_acc, 1), W)

    zpad = jnp.zeros((n_acc - n,), jnp.float32)
    q_p = jnp.concatenate([q[:, 0], zpad])
    base_p = jnp.concatenate([base[:, 0], zpad])
    invd_p = jnp.concatenate([invd[:, 0], zpad])
    rows_per_tile = n_acc // NSUB
    scalar_kernel = pl.kernel(
        _sc_scalar_pass(n, n_acc, half),
        out_type=jax.ShapeDtypeStruct((n_acc,), jnp.float32),
        mesh=mesh,
        scratch_types=[
            pltpu.MemorySpace.VMEM_SHARED((n_acc,), jnp.float32),
            pltpu.VMEM((half, CHUNK), jnp.int32),
            pltpu.VMEM((half, CHUNK), jnp.int32),
            pltpu.VMEM((CHUNK,), jnp.float32),
            pltpu.VMEM((CHUNK,), jnp.float32),
            pltpu.VMEM((CHUNK,), jnp.float32),
            pltpu.VMEM((rows_per_tile,), jnp.float32),
            pltpu.VMEM((rows_per_tile,), jnp.float32),
            pltpu.VMEM((rows_per_tile,), jnp.float32),
            pltpu.VMEM((rows_per_tile,), jnp.float32),
            pltpu.SemaphoreType.DMA((3,)),
            pltpu.SemaphoreType.DMA((3,)),
        ],
    )
    score = scalar_kernel(q_p, src_r, dst_r, base_p, invd_p)
    return score[:n, None]
